# Initial kernel scaffold; baseline (speedup 1.0000x reference)
import jax
import jax.numpy as jnp
from jax.experimental import pallas as pl


def kernel(hidden_states, expert_bias, W):
    raise NotImplementedError("write your pallas kernel here")


# speedup vs baseline: 1.1378x; 1.1378x over previous
import functools

import jax
import jax.numpy as jnp
from jax import lax
from jax.experimental import pallas as pl
from jax.experimental.pallas import tpu as pltpu
from jax.experimental.pallas import tpu_sc as plsc

B, S, D, E, K = 4, 8192, 4096, 64, 8
T = B * S
ROUTE_SCALE = 2.5

NC, NS = 2, 16
NW = NC * NS
CHUNK = T // NW
SUB = 512
LANES = 16

BT = 512


def _mm_body(x_ref, wt_ref, out_ref):
    out_ref[...] = lax.dot_general(
        x_ref[...], wt_ref[...], (((1,), (0,)), ((), ())),
        preferred_element_type=jnp.float32,
        precision=lax.Precision.DEFAULT,
    )


def _tc_logits(x, wt):
    return pl.pallas_call(
        _mm_body,
        grid=(T // BT,),
        in_specs=[
            pl.BlockSpec((BT, D), lambda i: (i, 0)),
            pl.BlockSpec((D, E), lambda i: (0, 0)),
        ],
        out_specs=pl.BlockSpec((BT, E), lambda i: (i, 0)),
        out_shape=jax.ShapeDtypeStruct((T, E), jnp.float32),
    )(x, wt)


def _sc_router_body(logits_hbm, bias_hbm, scores_hbm, sel_hbm,
                    logits_v, bias_v, scores_v, sel_v):
    wid = lax.axis_index("s") * NC + lax.axis_index("c")
    pltpu.sync_copy(bias_hbm, bias_v)

    iota = lax.iota(jnp.int32, LANES)
    mask8 = iota < K

    def merge(ak, av, bk, bv):
        rbk = lax.rev(bk, (0,))
        rbv = lax.rev(bv, (0,))
        m = ak >= rbk
        mk = jnp.where(m, ak, rbk)
        mv = jnp.where(m, av, rbv)
        return plsc.sort_key_val(mk, mv, descending=True)

    def row(r, carry):
        ks, vs = [], []
        for i in range(E // LANES):
            l = logits_v[r, pl.ds(LANES * i, LANES)]
            s = 1.0 / (1.0 + jnp.exp(-l))
            b = s + bias_v[pl.ds(LANES * i, LANES)]
            k_, v_ = plsc.sort_key_val(b, iota + LANES * i, descending=True)
            ks.append(k_)
            vs.append(v_)
        k01, v01 = merge(ks[0], vs[0], ks[1], vs[1])
        k23, v23 = merge(ks[2], vs[2], ks[3], vs[3])
        kt, vt = merge(k01, v01, k23, v23)
        bsel = plsc.load_gather(bias_v, [vt])
        raw = kt - bsel
        denom = jnp.sum(jnp.where(mask8, raw, jnp.zeros_like(raw)))
        denom_v = lax.broadcast(denom + 1e-20, (LANES,))
        norm = (raw * ROUTE_SCALE) / denom_v
        scores_v[r, :] = norm
        sel_v[r, :] = vt
        return carry

    def subchunk(j, carry):
        base = wid * CHUNK + j * SUB
        pltpu.sync_copy(logits_hbm.at[pl.ds(base, SUB)], logits_v)
        lax.fori_loop(0, SUB, row, 0)
        pltpu.sync_copy(scores_v, scores_hbm.at[pl.ds(base, SUB)])
        pltpu.sync_copy(sel_v, sel_hbm.at[pl.ds(base, SUB)])
        return carry

    lax.fori_loop(0, CHUNK // SUB, subchunk, 0)


@functools.partial(
    pl.kernel,
    out_type=(
        jax.ShapeDtypeStruct((T, LANES), jnp.float32),
        jax.ShapeDtypeStruct((T, LANES), jnp.int32),
    ),
    mesh=plsc.VectorSubcoreMesh(core_axis_name="c", subcore_axis_name="s"),
    scratch_types=[
        pltpu.VMEM((SUB, E), jnp.float32),
        pltpu.VMEM((E,), jnp.float32),
        pltpu.VMEM((SUB, LANES), jnp.float32),
        pltpu.VMEM((SUB, LANES), jnp.int32),
    ],
    compiler_params=pltpu.CompilerParams(
        needs_layout_passes=False, use_tc_tiling_on_sc=False),
)
def _sc_router(logits_hbm, bias_hbm, scores_hbm, sel_hbm,
               logits_v, bias_v, scores_v, sel_v):
    _sc_router_body(logits_hbm, bias_hbm, scores_hbm, sel_hbm,
                    logits_v, bias_v, scores_v, sel_v)


def kernel(hidden_states, expert_bias, W):
    x = hidden_states.reshape(-1, D)
    logits = _tc_logits(x, W.T)
    scores16, sel16 = _sc_router(logits, expert_bias)
    return logits, scores16[:, :K], sel16[:, :K]


# speedup vs baseline: 1.4458x; 1.2707x over previous
import functools

import jax
import jax.numpy as jnp
from jax import lax
from jax.experimental import pallas as pl
from jax.experimental.pallas import tpu as pltpu
from jax.experimental.pallas import tpu_sc as plsc

B, S, D, E, K = 4, 8192, 4096, 64, 8
T = B * S
ROUTE_SCALE = 2.5

NC, NS = 2, 16
NW = NC * NS
CHUNK = T // NW
SUB = 512
LANES = 16

BT = 512


def _mm_body(x_ref, wt_ref, out_ref):
    out_ref[...] = lax.dot_general(
        x_ref[...], wt_ref[...], (((1,), (0,)), ((), ())),
        preferred_element_type=jnp.float32,
        precision=lax.Precision.DEFAULT,
    )


def _tc_logits(x, wt):
    return pl.pallas_call(
        _mm_body,
        grid=(T // BT,),
        in_specs=[
            pl.BlockSpec((BT, D), lambda i: (i, 0)),
            pl.BlockSpec((D, E), lambda i: (0, 0)),
        ],
        out_specs=pl.BlockSpec((BT, E), lambda i: (i, 0)),
        out_shape=jax.ShapeDtypeStruct((T, E), jnp.float32),
    )(x, wt)


def _sc_router_body(logits_hbm, bias_hbm, scores_hbm, sel_hbm,
                    logits_v, bias_v, scores_v, sel_v):
    wid = lax.axis_index("s") * NC + lax.axis_index("c")
    pltpu.sync_copy(bias_hbm, bias_v)

    iota = lax.iota(jnp.int32, LANES)
    mask8 = iota < K

    def merge(ak, av, bk, bv):
        rbk = lax.rev(bk, (0,))
        rbv = lax.rev(bv, (0,))
        m = ak >= rbk
        mk = jnp.where(m, ak, rbk)
        mv = jnp.where(m, av, rbv)
        return plsc.sort_key_val(mk, mv, descending=True)

    def row(r):
        ks, vs = [], []
        for i in range(E // LANES):
            l = logits_v[r, pl.ds(LANES * i, LANES)]
            s = 1.0 / (1.0 + jnp.exp(-l))
            b = s + bias_v[pl.ds(LANES * i, LANES)]
            k_, v_ = plsc.sort_key_val(b, iota + LANES * i, descending=True)
            ks.append(k_)
            vs.append(v_)
        k01, v01 = merge(ks[0], vs[0], ks[1], vs[1])
        k23, v23 = merge(ks[2], vs[2], ks[3], vs[3])
        kt, vt = merge(k01, v01, k23, v23)
        bsel = plsc.load_gather(bias_v, [vt])
        raw = kt - bsel
        denom = jnp.sum(jnp.where(mask8, raw, jnp.zeros_like(raw)))
        denom_v = lax.broadcast(denom + 1e-20, (LANES,))
        norm = (raw * ROUTE_SCALE) / denom_v
        scores_v[r, :] = norm
        sel_v[r, :] = vt

    def subchunk(j, carry):
        base = wid * CHUNK + j * SUB
        pltpu.sync_copy(logits_hbm.at[pl.ds(base, SUB)], logits_v)
        plsc.parallel_loop(0, SUB, unroll=4)(row)
        pltpu.sync_copy(scores_v, scores_hbm.at[pl.ds(base, SUB)])
        pltpu.sync_copy(sel_v, sel_hbm.at[pl.ds(base, SUB)])
        return carry

    lax.fori_loop(0, CHUNK // SUB, subchunk, 0)


@functools.partial(
    pl.kernel,
    out_type=(
        jax.ShapeDtypeStruct((T, LANES), jnp.float32),
        jax.ShapeDtypeStruct((T, LANES), jnp.int32),
    ),
    mesh=plsc.VectorSubcoreMesh(core_axis_name="c", subcore_axis_name="s"),
    scratch_types=[
        pltpu.VMEM((SUB, E), jnp.float32),
        pltpu.VMEM((E,), jnp.float32),
        pltpu.VMEM((SUB, LANES), jnp.float32),
        pltpu.VMEM((SUB, LANES), jnp.int32),
    ],
    compiler_params=pltpu.CompilerParams(
        needs_layout_passes=False, use_tc_tiling_on_sc=False),
)
def _sc_router(logits_hbm, bias_hbm, scores_hbm, sel_hbm,
               logits_v, bias_v, scores_v, sel_v):
    _sc_router_body(logits_hbm, bias_hbm, scores_hbm, sel_hbm,
                    logits_v, bias_v, scores_v, sel_v)


def kernel(hidden_states, expert_bias, W):
    x = hidden_states.reshape(-1, D)
    logits = _tc_logits(x, W.T)
    scores16, sel16 = _sc_router(logits, expert_bias)
    return logits, scores16[:, :K], sel16[:, :K]
